# Initial kernel scaffold; baseline (speedup 1.0000x reference)
import jax
import jax.numpy as jnp
from jax.experimental import pallas as pl


def kernel(x, U, V, indices):
    raise NotImplementedError("write your pallas kernel here")


# speedup vs baseline: 23.0586x; 23.0586x over previous
import functools

import jax
import jax.numpy as jnp
from jax import lax
from jax.experimental import pallas as pl
from jax.experimental.pallas import tpu as pltpu
from jax.experimental.pallas import tpu_sc as plsc

N_PRE = 100000
N_PRE_PAD = 100096
N_POST = 100000
KK = 32
RR = 16
BB = 8
L = 16
NW = 32
ROWS_W = 3200
N_POST_PAD = NW * ROWS_W
WIN = 1792
GROUPS = ROWS_W // L


def _sc_body(vt_hbm, ut_hbm, x_hbm, idxt_hbm, y_hbm,
             vt_win, x_win, u_buf, idx_buf, y_buf):
    wid = lax.axis_index("s") * 2 + lax.axis_index("c")
    nbase = wid * ROWS_W

    pltpu.sync_copy(ut_hbm.at[:, pl.ds(nbase, ROWS_W)], u_buf)

    def zero_body(g, c):
        z = jnp.zeros((L,), jnp.float32)
        for b in range(BB):
            y_buf[b, pl.ds(g * L, L)] = z
        return c
    lax.fori_loop(0, GROUPS, zero_body, 0)

    def j_body(j, c):
        base_floor = (j * (N_PRE - 1)) // (KK - 1)
        lo = jnp.minimum(
            jnp.maximum(base_floor - 782, 0) & (-128), N_PRE_PAD - WIN)
        lo = pl.multiple_of(lo, 128)
        for r in range(RR):
            pltpu.sync_copy(vt_hbm.at[r, pl.ds(lo, WIN)],
                            vt_win.at[pl.ds(r * WIN, WIN)])
        for b in range(BB):
            pltpu.sync_copy(x_hbm.at[b, pl.ds(lo, WIN)],
                            x_win.at[pl.ds(b * WIN, WIN)])
        pltpu.sync_copy(idxt_hbm.at[pl.ds(j * N_POST_PAD + nbase, ROWS_W)],
                        idx_buf)

        def g_body(g, cc):
            g16 = g * L
            iv = idx_buf[pl.ds(g16, L)]
            il = jnp.minimum(jnp.maximum(iv - lo, 0), WIN - 1)
            val = jnp.zeros((L,), jnp.float32)
            for r in range(RR):
                vr = plsc.load_gather(vt_win, [il + (r * WIN)])
                ur = u_buf[r, pl.ds(g16, L)]
                val = val + vr * ur
            for b in range(BB):
                xb = plsc.load_gather(x_win, [il + (b * WIN)])
                plsc.addupdate(y_buf.at[b, pl.ds(g16, L)], xb * val)
            return cc
        lax.fori_loop(0, GROUPS, g_body, 0)
        return c
    lax.fori_loop(0, KK, j_body, 0)

    pltpu.sync_copy(y_buf, y_hbm.at[:, pl.ds(nbase, ROWS_W)])


_sc_call = functools.partial(
    pl.kernel,
    out_type=jax.ShapeDtypeStruct((BB, N_POST_PAD), jnp.float32),
    mesh=plsc.VectorSubcoreMesh(core_axis_name="c", subcore_axis_name="s"),
    compiler_params=pltpu.CompilerParams(
        use_tc_tiling_on_sc=False, needs_layout_passes=False),
    scratch_types=[
        pltpu.VMEM((RR * WIN,), jnp.float32),
        pltpu.VMEM((BB * WIN,), jnp.float32),
        pltpu.VMEM((RR, ROWS_W), jnp.float32),
        pltpu.VMEM((ROWS_W,), jnp.int32),
        pltpu.VMEM((BB, ROWS_W), jnp.float32),
    ],
)(_sc_body)


@jax.jit
def _run(x, U, V, indices):
    pad_n = N_POST_PAD - N_POST
    pad_p = N_PRE_PAD - N_PRE
    vt = jnp.pad(V.T, ((0, 0), (0, pad_p)))
    xp = jnp.pad(x, ((0, 0), (0, pad_p)))
    ut = jnp.pad(U.T, ((0, 0), (0, pad_n)))
    idxt = jnp.pad(indices.reshape(N_POST, KK).T,
                   ((0, 0), (0, pad_n))).reshape(-1)
    ypad = _sc_call(vt, ut, xp, idxt)
    return ypad[:, :N_POST]


def kernel(x, U, V, indices):
    return _run(x, U, V, indices)


# speedup vs baseline: 37.0032x; 1.6047x over previous
import functools

import jax
import jax.numpy as jnp
from jax import lax
from jax.experimental import pallas as pl
from jax.experimental.pallas import tpu as pltpu
from jax.experimental.pallas import tpu_sc as plsc

N_PRE = 100000
N_PRE_PAD = 100096
N_POST = 100000
KK = 32
RR = 16
BB = 8
L = 16
NW = 32
ROWS_W = 3200
N_POST_PAD = NW * ROWS_W
WIN = 1792
GROUPS = ROWS_W // L


def _sc_body(vt_hbm, ut_hbm, x_hbm, idxt_hbm, y_hbm,
             vt_win, x_win, u_buf, idx_buf, y_buf, dma_sem):
    wid = lax.axis_index("s") * 2 + lax.axis_index("c")
    nbase = wid * ROWS_W

    pltpu.sync_copy(ut_hbm.at[:, pl.ds(nbase, ROWS_W)], u_buf)

    def zero_body(g, c):
        z = jnp.zeros((L,), jnp.float32)
        for b in range(BB):
            y_buf[b, pl.ds(g * L, L)] = z
        return c
    lax.fori_loop(0, GROUPS, zero_body, 0)

    def j_body(j, c):
        base_floor = (j * (N_PRE - 1)) // (KK - 1)
        lo = jnp.minimum(
            jnp.maximum(base_floor - 782, 0) & (-128), N_PRE_PAD - WIN)
        lo = pl.multiple_of(lo, 128)
        copies = []
        for r in range(RR):
            copies.append(pltpu.make_async_copy(
                vt_hbm.at[r, pl.ds(lo, WIN)],
                vt_win.at[pl.ds(r * WIN, WIN)], dma_sem))
        for b in range(BB):
            copies.append(pltpu.make_async_copy(
                x_hbm.at[b, pl.ds(lo, WIN)],
                x_win.at[pl.ds(b * WIN, WIN)], dma_sem))
        copies.append(pltpu.make_async_copy(
            idxt_hbm.at[pl.ds(j * N_POST_PAD + nbase, ROWS_W)],
            idx_buf, dma_sem))
        for cp in copies:
            cp.start()
        for cp in copies:
            cp.wait()

        def g_body(g, cc):
            g16 = g * L
            iv = idx_buf[pl.ds(g16, L)]
            il = jnp.minimum(jnp.maximum(iv - lo, 0), WIN - 1)
            val = jnp.zeros((L,), jnp.float32)
            for r in range(RR):
                vr = plsc.load_gather(vt_win, [il + (r * WIN)])
                ur = u_buf[r, pl.ds(g16, L)]
                val = val + vr * ur
            for b in range(BB):
                xb = plsc.load_gather(x_win, [il + (b * WIN)])
                plsc.addupdate(y_buf.at[b, pl.ds(g16, L)], xb * val)
            return cc
        lax.fori_loop(0, GROUPS, g_body, 0)
        return c
    lax.fori_loop(0, KK, j_body, 0)

    pltpu.sync_copy(y_buf, y_hbm.at[:, pl.ds(nbase, ROWS_W)])


_sc_call = functools.partial(
    pl.kernel,
    out_type=jax.ShapeDtypeStruct((BB, N_POST_PAD), jnp.float32),
    mesh=plsc.VectorSubcoreMesh(core_axis_name="c", subcore_axis_name="s"),
    compiler_params=pltpu.CompilerParams(
        use_tc_tiling_on_sc=False, needs_layout_passes=False),
    scratch_types=[
        pltpu.VMEM((RR * WIN,), jnp.float32),
        pltpu.VMEM((BB * WIN,), jnp.float32),
        pltpu.VMEM((RR, ROWS_W), jnp.float32),
        pltpu.VMEM((ROWS_W,), jnp.int32),
        pltpu.VMEM((BB, ROWS_W), jnp.float32),
        pltpu.SemaphoreType.DMA,
    ],
)(_sc_body)


@jax.jit
def _run(x, U, V, indices):
    pad_n = N_POST_PAD - N_POST
    pad_p = N_PRE_PAD - N_PRE
    vt = jnp.pad(V.T, ((0, 0), (0, pad_p)))
    xp = jnp.pad(x, ((0, 0), (0, pad_p)))
    ut = jnp.pad(U.T, ((0, 0), (0, pad_n)))
    idxt = jnp.pad(indices.reshape(N_POST, KK).T,
                   ((0, 0), (0, pad_n))).reshape(-1)
    ypad = _sc_call(vt, ut, xp, idxt)
    return ypad[:, :N_POST]


def kernel(x, U, V, indices):
    return _run(x, U, V, indices)
